# baseline (speedup 1.0000x reference)
import jax
import jax.numpy as jnp
from jax import lax
from jax.experimental import pallas as pl
from jax.experimental.pallas import tpu as pltpu
from jax.experimental.pallas import tpu_sc as plsc

IN_D = 128
OUT_D = 512
NC = 2
NS = 16
NW = NC * NS
R = 64
L = 16


def _sc_body(x_hbm, fidx_hbm, out_hbm, fidx_v, xin, outb):
    n = x_hbm.shape[0] // IN_D
    rw = n // NW
    wid = lax.axis_index("s") * NC + lax.axis_index("c")
    pltpu.sync_copy(fidx_hbm, fidx_v)

    def sub_block(t, carry):
        base = (wid * rw + t * R) * IN_D
        obase = (wid * rw + t * R) * OUT_D
        pltpu.sync_copy(x_hbm.at[pl.ds(base, R * IN_D)], xin)
        for cg in range(OUT_D // L):
            cvec = fidx_v[pl.ds(cg * L, L)]

            def row_body(r, gvec):
                v = plsc.load_gather(xin, [gvec])
                outb[pl.ds(r * OUT_D + cg * L, L)] = v
                return gvec + IN_D

            lax.fori_loop(0, R, row_body, cvec, unroll=False)
        pltpu.sync_copy(outb, out_hbm.at[pl.ds(obase, R * OUT_D)])
        return carry

    lax.fori_loop(0, rw // R, sub_block, 0, unroll=False)


def kernel(x, indices):
    n = x.shape[0]
    fidx = jnp.concatenate(
        [jnp.arange(IN_D, dtype=jnp.int32), indices.astype(jnp.int32)]
    )
    mesh = plsc.VectorSubcoreMesh(core_axis_name="c", subcore_axis_name="s")
    run = pl.kernel(
        _sc_body,
        out_type=jax.ShapeDtypeStruct((n * OUT_D,), jnp.float32),
        mesh=mesh,
        scratch_types=[
            pltpu.VMEM((OUT_D,), jnp.int32),
            pltpu.VMEM((R * IN_D,), jnp.float32),
            pltpu.VMEM((R * OUT_D,), jnp.float32),
        ],
        compiler_params=pltpu.CompilerParams(needs_layout_passes=False),
    )
    return run(x.reshape(-1), fidx).reshape(n, OUT_D)


# speedup vs baseline: 1.1179x; 1.1179x over previous
import jax
import jax.numpy as jnp
from jax import lax
from jax.experimental import pallas as pl
from jax.experimental.pallas import tpu as pltpu
from jax.experimental.pallas import tpu_sc as plsc

IN_D = 128
OUT_D = 512
NC = 2
NS = 16
NW = NC * NS
R = 64
L = 16
NG = OUT_D // L


def _sc_body(x_hbm, fidx_hbm, out_hbm, fidx_v, xin, outb, is0, is1, os0, os1):
    n = x_hbm.shape[0] // IN_D
    rw = n // NW
    nt = rw // R
    wid = lax.axis_index("s") * NC + lax.axis_index("c")
    pltpu.sync_copy(fidx_hbm, fidx_v)
    cvecs = [fidx_v[pl.ds(g * L, L)] for g in range(NG)]
    isems = (is0, is1)
    osems = (os0, os1)

    def in_copy(t):
        return pltpu.make_async_copy(
            x_hbm.at[pl.ds((wid * rw + t * R) * IN_D, R * IN_D)],
            xin.at[pl.ds((t % 2) * R * IN_D, R * IN_D)],
            isems[t % 2],
        )

    def out_copy(t):
        return pltpu.make_async_copy(
            outb.at[pl.ds((t % 2) * R * OUT_D, R * OUT_D)],
            out_hbm.at[pl.ds((wid * rw + t * R) * OUT_D, R * OUT_D)],
            osems[t % 2],
        )

    in_copy(0).start()
    for t in range(nt):
        b = t % 2
        if t + 1 < nt:
            in_copy(t + 1).start()
        in_copy(t).wait()
        if t >= 2:
            out_copy(t - 2).wait()

        def row_body(r, _, b=b):
            src = xin.at[pl.ds(b * R * IN_D + r * IN_D, IN_D)]
            obase = b * R * OUT_D + r * OUT_D
            for g in range(NG):
                outb[pl.ds(obase + g * L, L)] = plsc.load_gather(
                    src, [cvecs[g]]
                )
            return 0

        lax.fori_loop(0, R, row_body, 0, unroll=False)
        out_copy(t).start()
    out_copy(nt - 2).wait()
    out_copy(nt - 1).wait()


def kernel(x, indices):
    n = x.shape[0]
    fidx = jnp.concatenate(
        [jnp.arange(IN_D, dtype=jnp.int32), indices.astype(jnp.int32)]
    )
    mesh = plsc.VectorSubcoreMesh(core_axis_name="c", subcore_axis_name="s")
    run = pl.kernel(
        _sc_body,
        out_type=jax.ShapeDtypeStruct((n * OUT_D,), jnp.float32),
        mesh=mesh,
        scratch_types=[
            pltpu.VMEM((OUT_D,), jnp.int32),
            pltpu.VMEM((2 * R * IN_D,), jnp.float32),
            pltpu.VMEM((2 * R * OUT_D,), jnp.float32),
            pltpu.SemaphoreType.DMA,
            pltpu.SemaphoreType.DMA,
            pltpu.SemaphoreType.DMA,
            pltpu.SemaphoreType.DMA,
        ],
        compiler_params=pltpu.CompilerParams(needs_layout_passes=False),
    )
    return run(x.reshape(-1), fidx).reshape(n, OUT_D)


# speedup vs baseline: 1.7613x; 1.5754x over previous
import jax
import jax.numpy as jnp
from jax import lax
from jax.experimental import pallas as pl
from jax.experimental.pallas import tpu as pltpu
from jax.experimental.pallas import tpu_sc as plsc

IN_D = 128
OUT_D = 512
NC = 2
NS = 16
NW = NC * NS
R = 64
L = 16
NG = OUT_D // L


def _sc_body(x_hbm, fidx_hbm, out_hbm, fidx_v, xin, outb, is0, is1, os0, os1):
    n = x_hbm.shape[0] // IN_D
    rw = n // NW
    nt = rw // R
    wid = lax.axis_index("s") * NC + lax.axis_index("c")
    pltpu.sync_copy(fidx_hbm, fidx_v)
    cvecs = [fidx_v[pl.ds(g * L, L)] for g in range(NG)]
    isems = (is0, is1)
    osems = (os0, os1)

    def in_copy(t):
        return pltpu.make_async_copy(
            x_hbm.at[pl.ds((wid * rw + t * R) * IN_D, R * IN_D)],
            xin.at[pl.ds((t % 2) * R * IN_D, R * IN_D)],
            isems[t % 2],
        )

    def out_copy(t):
        return pltpu.make_async_copy(
            outb.at[pl.ds((t % 2) * R * OUT_D, R * OUT_D)],
            out_hbm.at[pl.ds((wid * rw + t * R) * OUT_D, R * OUT_D)],
            osems[t % 2],
        )

    in_copy(0).start()
    for t in range(nt):
        b = t % 2
        if t + 1 < nt:
            in_copy(t + 1).start()
        in_copy(t).wait()
        if t >= 2:
            out_copy(t - 2).wait()

        def row_body(r, _, b=b):
            src = xin.at[pl.ds(b * R * IN_D + r * IN_D, IN_D)]
            obase = b * R * OUT_D + r * OUT_D
            for g0 in range(0, NG, 8):
                vals = [
                    plsc.load_gather(src, [cvecs[g]])
                    for g in range(g0, g0 + 8)
                ]
                for j in range(8):
                    outb[pl.ds(obase + (g0 + j) * L, L)] = vals[j]
            return 0

        lax.fori_loop(0, R, row_body, 0, unroll=False)
        out_copy(t).start()
    out_copy(nt - 2).wait()
    out_copy(nt - 1).wait()


def kernel(x, indices):
    n = x.shape[0]
    fidx = jnp.concatenate(
        [jnp.arange(IN_D, dtype=jnp.int32), indices.astype(jnp.int32)]
    )
    mesh = plsc.VectorSubcoreMesh(core_axis_name="c", subcore_axis_name="s")
    run = pl.kernel(
        _sc_body,
        out_type=jax.ShapeDtypeStruct((n * OUT_D,), jnp.float32),
        mesh=mesh,
        scratch_types=[
            pltpu.VMEM((OUT_D,), jnp.int32),
            pltpu.VMEM((2 * R * IN_D,), jnp.float32),
            pltpu.VMEM((2 * R * OUT_D,), jnp.float32),
            pltpu.SemaphoreType.DMA,
            pltpu.SemaphoreType.DMA,
            pltpu.SemaphoreType.DMA,
            pltpu.SemaphoreType.DMA,
        ],
        compiler_params=pltpu.CompilerParams(needs_layout_passes=False),
    )
    return run(x.reshape(-1), fidx).reshape(n, OUT_D)


# speedup vs baseline: 3.1288x; 1.7765x over previous
import jax
import jax.numpy as jnp
from jax import lax
from jax.experimental import pallas as pl
from jax.experimental.pallas import tpu as pltpu
from jax.experimental.pallas import tpu_sc as plsc

IN_D = 128
OUT_D = 512
NC = 2
NS = 16
NW = NC * NS
R = 64
L = 16
NG = OUT_D // L


def _sc_body(x_hbm, fidx_hbm, out_hbm, fidx_v, xin, outb, is0, is1, os0, os1):
    n = x_hbm.shape[0]
    rw = n // NW
    nt = rw // R
    wid = lax.axis_index("s") * NC + lax.axis_index("c")
    pltpu.sync_copy(fidx_hbm, fidx_v)
    cvecs = [fidx_v[pl.ds(g * L, L)] for g in range(NG)]
    isems = (is0, is1)
    osems = (os0, os1)

    def in_copy(t):
        return pltpu.make_async_copy(
            x_hbm.at[pl.ds(wid * rw + t * R, R)],
            xin.at[pl.ds((t % 2) * R, R)],
            isems[t % 2],
        )

    def out_copy(t):
        return pltpu.make_async_copy(
            outb.at[pl.ds((t % 2) * R, R)],
            out_hbm.at[pl.ds(wid * rw + t * R, R)],
            osems[t % 2],
        )

    in_copy(0).start()
    for t in range(nt):
        b = t % 2
        if t + 1 < nt:
            in_copy(t + 1).start()
        in_copy(t).wait()
        if t >= 2:
            out_copy(t - 2).wait()

        def row_body(r, _, b=b):
            row = b * R + r
            rvec = jnp.full((L,), row, dtype=jnp.int32)
            for g0 in range(0, NG, 8):
                vals = [
                    plsc.load_gather(xin, [rvec, cvecs[g]])
                    for g in range(g0, g0 + 8)
                ]
                for j in range(8):
                    outb[row, pl.ds((g0 + j) * L, L)] = vals[j]
            return 0

        lax.fori_loop(0, R, row_body, 0, unroll=False)
        out_copy(t).start()
    out_copy(nt - 2).wait()
    out_copy(nt - 1).wait()


def kernel(x, indices):
    n = x.shape[0]
    fidx = jnp.concatenate(
        [jnp.arange(IN_D, dtype=jnp.int32), indices.astype(jnp.int32)]
    )
    mesh = plsc.VectorSubcoreMesh(core_axis_name="c", subcore_axis_name="s")
    run = pl.kernel(
        _sc_body,
        out_type=jax.ShapeDtypeStruct((n, OUT_D), jnp.float32),
        mesh=mesh,
        scratch_types=[
            pltpu.VMEM((OUT_D,), jnp.int32),
            pltpu.VMEM((2 * R, IN_D), jnp.float32),
            pltpu.VMEM((2 * R, OUT_D), jnp.float32),
            pltpu.SemaphoreType.DMA,
            pltpu.SemaphoreType.DMA,
            pltpu.SemaphoreType.DMA,
            pltpu.SemaphoreType.DMA,
        ],
        compiler_params=pltpu.CompilerParams(needs_layout_passes=False),
    )
    return run(x, fidx)
